# Initial kernel scaffold; baseline (speedup 1.0000x reference)
import jax
import jax.numpy as jnp
from jax.experimental import pallas as pl


def kernel(x, edge_index, mask, W1, as1, ad1, b1, W2, as2, ad2, b2, W3, as3, ad3, b3, pool_W, pool_b, Wih_f, Whh_f, bih_f, bhh_f, Wih_r, Whh_r, bih_r, bhh_r, Wmu, bmu, Wlv, blv, Wpi, bpi):
    raise NotImplementedError("write your pallas kernel here")


# speedup vs baseline: 101.2052x; 101.2052x over previous
import functools

import jax
import jax.numpy as jnp
from jax import lax
from jax.experimental import pallas as pl
from jax.experimental.pallas import tpu as pltpu
from jax.experimental.pallas import tpu_sc as plsc

F32 = jnp.float32
I32 = jnp.int32

B, T, N, FD = 4, 8, 1000, 128
HID, HEADS, RNN, LAT, K = 32, 4, 128, 64, 32
G = B * T
NP = 1008
NE = 16000
CHE = 320
NCH = NE // CHE
ECH = 640
ENCH = NE // ECH
PREC = None


def _prep_common(x, W_ref, as_ref, ad_ref, heads):
    xw = jnp.dot(x, W_ref[...], preferred_element_type=F32, precision=PREC)
    asrs, adss, wselfs, mparts = [], [], [], []
    for h in range(heads):
        xwh = xw[:, h * HID:(h + 1) * HID]
        asr = jnp.sum(xwh * as_ref[h][None, :], axis=-1)
        ads = jnp.sum(xwh * ad_ref[h][None, :], axis=-1)
        e = asr + ads
        w = jnp.exp(jnp.maximum(e, 0.2 * e))
        asrs.append(asr)
        adss.append(ads)
        wselfs.append(w)
        mparts.append(xwh * w[:, None])
    return xw, asrs, adss, wselfs, jnp.concatenate(mparts, axis=-1)


def _normalize(msg_ref, den_ref, bias_ref, r, heads_in):
    parts = []
    for hh in range(heads_in):
        m = msg_ref[r][:, hh * HID:(hh + 1) * HID]
        dn = den_ref[r, hh]
        parts.append(jnp.maximum(m / dn[:, None] + bias_ref[hh][None, :], 0.0))
    return jnp.concatenate(parts, axis=-1)


def _prep1_body(x_ref, W_ref, as_ref, ad_ref, xw_ref, asrc_ref, adst_ref,
                msgi_ref, deni_ref, *, heads):
    xw, asrs, adss, wselfs, msgi = _prep_common(
        x_ref[0, 0], W_ref, as_ref, ad_ref, heads)
    xw_ref[0] = xw
    msgi_ref[0] = msgi
    for h in range(heads):
        asrc_ref[h, 0] = asrs[h]
        adst_ref[h, 0] = adss[h]
        deni_ref[h, 0] = wselfs[h]


def _prepL_body(msg_ref, den_ref, bias_ref, W_ref, as_ref, ad_ref, xw_ref,
                asrc_ref, adst_ref, msgi_ref, deni_ref, *, heads, heads_in):
    x = _normalize(msg_ref, den_ref, bias_ref, 0, heads_in)
    xw, asrs, adss, wselfs, msgi = _prep_common(x, W_ref, as_ref, ad_ref,
                                                heads)
    xw_ref[0] = xw
    msgi_ref[0] = msgi
    for h in range(heads):
        asrc_ref[h, 0] = asrs[h]
        adst_ref[h, 0] = adss[h]
        deni_ref[h, 0] = wselfs[h]


def _prep3_body(msg_ref, den_ref, bias_ref, W_ref, as_ref, ad_ref, xw_ref,
                asrc_ref, adst_ref, msgi_ref, deni_ref, *, heads_in):
    for r in range(4):
        x = _normalize(msg_ref, den_ref, bias_ref, r, heads_in)
        xw, asrs, adss, wselfs, msgi = _prep_common(x, W_ref, as_ref, ad_ref,
                                                    1)
        xw_ref[0, :, r * HID:(r + 1) * HID] = xw
        msgi_ref[0, :, r * HID:(r + 1) * HID] = msgi
        asrc_ref[r, 0] = asrs[0]
        adst_ref[r, 0] = adss[0]
        deni_ref[r, 0] = wselfs[0]


def _prep_outs(heads):
    P = G * heads
    out_shapes = (
        jax.ShapeDtypeStruct((G, NP, heads * HID), F32),
        jax.ShapeDtypeStruct((P, 1, NP), F32),
        jax.ShapeDtypeStruct((P, 1, NP), F32),
        jax.ShapeDtypeStruct((G, NP, heads * HID), F32),
        jax.ShapeDtypeStruct((P, 1, NP), F32),
    )
    xw_spec = pl.BlockSpec((1, NP, heads * HID), lambda g: (g, 0, 0))
    pv_spec = pl.BlockSpec((heads, 1, NP), lambda g: (g, 0, 0))
    out_specs = (xw_spec, pv_spec, pv_spec, xw_spec, pv_spec)
    return out_shapes, out_specs


def _prep1(x_p, W, a_s, a_d, heads):
    out_shapes, out_specs = _prep_outs(heads)
    fin = W.shape[0]
    return pl.pallas_call(
        functools.partial(_prep1_body, heads=heads),
        grid=(G,),
        in_specs=[
            pl.BlockSpec((1, 1, NP, fin), lambda g: (g, 0, 0, 0)),
            pl.BlockSpec((fin, heads * HID), lambda g: (0, 0)),
            pl.BlockSpec((heads, HID), lambda g: (0, 0)),
            pl.BlockSpec((heads, HID), lambda g: (0, 0)),
        ],
        out_specs=out_specs,
        out_shape=out_shapes,
    )(x_p, W, a_s, a_d)


def _prepL(msg, den, bias, W, a_s, a_d, heads, heads_in):
    out_shapes, out_specs = _prep_outs(heads)
    fin = W.shape[0]
    return pl.pallas_call(
        functools.partial(_prepL_body, heads=heads, heads_in=heads_in),
        grid=(G,),
        in_specs=[
            pl.BlockSpec((1, NP, heads_in * HID), lambda g: (g, 0, 0)),
            pl.BlockSpec((1, heads_in, NP), lambda g: (g, 0, 0)),
            pl.BlockSpec((heads_in, HID), lambda g: (0, 0)),
            pl.BlockSpec((fin, heads * HID), lambda g: (0, 0)),
            pl.BlockSpec((heads, HID), lambda g: (0, 0)),
            pl.BlockSpec((heads, HID), lambda g: (0, 0)),
        ],
        out_specs=out_specs,
        out_shape=out_shapes,
    )(msg, den, bias, W, a_s, a_d)


def _prep3(msg, den, bias, W, a_s, a_d, heads_in):
    out_shapes = (
        jax.ShapeDtypeStruct((G // 4, NP, 4 * HID), F32),
        jax.ShapeDtypeStruct((G, 1, NP), F32),
        jax.ShapeDtypeStruct((G, 1, NP), F32),
        jax.ShapeDtypeStruct((G // 4, NP, 4 * HID), F32),
        jax.ShapeDtypeStruct((G, 1, NP), F32),
    )
    xw_spec = pl.BlockSpec((1, NP, 4 * HID), lambda q: (q, 0, 0))
    pv_spec = pl.BlockSpec((4, 1, NP), lambda q: (q, 0, 0))
    out_specs = (xw_spec, pv_spec, pv_spec, xw_spec, pv_spec)
    fin = W.shape[0]
    return pl.pallas_call(
        functools.partial(_prep3_body, heads_in=heads_in),
        grid=(G // 4,),
        in_specs=[
            pl.BlockSpec((4, NP, heads_in * HID), lambda q: (q, 0, 0)),
            pl.BlockSpec((4, heads_in, NP), lambda q: (q, 0, 0)),
            pl.BlockSpec((heads_in, HID), lambda q: (0, 0)),
            pl.BlockSpec((fin, HID), lambda q: (0, 0)),
            pl.BlockSpec((1, HID), lambda q: (0, 0)),
            pl.BlockSpec((1, HID), lambda q: (0, 0)),
        ],
        out_specs=out_specs,
        out_shape=out_shapes,
    )(msg, den, bias, W, a_s, a_d)


def _edgew_body(s_ref, d_ref, asrc_ref, adst_ref, deni_ref, den_ref):
    ch = pl.program_id(0)
    sv = s_ref[0, 0]
    dv = d_ref[0, 0]
    nodes = lax.broadcasted_iota(I32, (ECH, NP), 1)
    oh_s = (sv[:, None] == nodes).astype(F32)
    oh_d = (dv[:, None] == nodes).astype(F32)
    asrc_e = lax.dot_general(asrc_ref[...], oh_s, (((1,), (1,)), ((), ())),
                             preferred_element_type=F32, precision=PREC)
    adst_e = lax.dot_general(adst_ref[...], oh_d, (((1,), (1,)), ((), ())),
                             preferred_element_type=F32, precision=PREC)
    e = asrc_e + adst_e
    w = jnp.exp(jnp.maximum(e, 0.2 * e))

    @pl.when(ch == 0)
    def _():
        den_ref[...] = deni_ref[...]

    den_ref[...] += lax.dot_general(w, oh_d, (((1,), (0,)), ((), ())),
                                    preferred_element_type=F32, precision=PREC)


def _edgew(s3, d3, asrc, adst, deni, P):
    return pl.pallas_call(
        _edgew_body,
        grid=(ENCH,),
        in_specs=[
            pl.BlockSpec((1, 1, ECH), lambda ch: (ch, 0, 0)),
            pl.BlockSpec((1, 1, ECH), lambda ch: (ch, 0, 0)),
            pl.BlockSpec((P, NP), lambda ch: (0, 0)),
            pl.BlockSpec((P, NP), lambda ch: (0, 0)),
            pl.BlockSpec((P, NP), lambda ch: (0, 0)),
        ],
        out_specs=pl.BlockSpec((P, NP), lambda ch: (0, 0)),
        out_shape=jax.ShapeDtypeStruct((P, NP), F32),
    )(s3, d3, asrc, adst, deni)


def _scale_rows(tmp_v, w_ch):
    def mrow(j, c2):
        base = j * 16
        wvecs = [w_ch[h, pl.ds(base, 16)] for h in range(4)]
        for lane in range(16):
            e = base + lane
            for h in range(4):
                ws = wvecs[h][lane]
                tmp_v[e, pl.ds(h * 32, 16)] = tmp_v[e, pl.ds(h * 32, 16)] * ws
                tmp_v[e, pl.ds(h * 32 + 16, 16)] = (
                    tmp_v[e, pl.ds(h * 32 + 16, 16)] * ws)
        return c2

    lax.fori_loop(0, CHE // 16, mrow, 0)


def _do_chunk(ch, g, xw_hbm, sf_hbm, df_hbm, msg_reg,
              s_ch, d_ch, w_ch, asrc_v, adst_v, tmp_v):
    pltpu.sync_copy(sf_hbm.at[pl.ds(ch * CHE, CHE)], s_ch)
    pltpu.sync_copy(df_hbm.at[pl.ds(ch * CHE, CHE)], d_ch)
    pltpu.sync_copy(xw_hbm.at[g].at[s_ch], tmp_v)

    def wgrp(kk, c2):
        sv = s_ch[pl.ds(kk * 16, 16)]
        dv = d_ch[pl.ds(kk * 16, 16)]
        for h in range(4):
            a1 = plsc.load_gather(asrc_v, [sv + h * NP])
            a2 = plsc.load_gather(adst_v, [dv + h * NP])
            e = a1 + a2
            w_ch[h, pl.ds(kk * 16, 16)] = jnp.exp(jnp.maximum(e, 0.2 * e))
        return c2

    lax.fori_loop(0, CHE // 16, wgrp, 0)
    _scale_rows(tmp_v, w_ch)
    pltpu.sync_copy(tmp_v, msg_reg.at[d_ch], add=True)


def _gat_sc12():
    mesh = plsc.VectorSubcoreMesh(core_axis_name="c", subcore_axis_name="s",
                                  num_cores=2, num_subcores=16)

    @functools.partial(
        pl.kernel,
        out_type=jax.ShapeDtypeStruct((G, NP, 4 * HID), F32),
        mesh=mesh,
        compiler_params=pltpu.CompilerParams(needs_layout_passes=False),
        scratch_types=[
            pltpu.VMEM_SHARED((8, NP, 4 * HID), F32),
            pltpu.VMEM((CHE,), I32),
            pltpu.VMEM((CHE,), I32),
            pltpu.VMEM((4, CHE), F32),
            pltpu.VMEM((4 * NP,), F32),
            pltpu.VMEM((4 * NP,), F32),
            pltpu.VMEM((CHE, 4 * HID), F32),
        ],
    )
    def k(xw_hbm, msgi_hbm, asrc_hbm, adst_hbm, sf_hbm, df_hbm, msg_out,
          msg_spm, s_ch, d_ch, w_ch, asrc_v, adst_v, tmp_v):
        cid = lax.axis_index("c")
        sid = lax.axis_index("s")
        reg = sid // 2
        half = sid % 2
        for pp in range(2):
            g = pp * 16 + cid * 8 + reg
            pltpu.sync_copy(asrc_hbm.at[g], asrc_v)
            pltpu.sync_copy(adst_hbm.at[g], adst_v)

            @pl.when(half == 0)
            def _():
                pltpu.sync_copy(msgi_hbm.at[g], msg_spm.at[reg])

            plsc.subcore_barrier()

            def mchunk(j, carry):
                _do_chunk(half * (NCH // 2) + j, g, xw_hbm, sf_hbm,
                          df_hbm, msg_spm.at[reg], s_ch, d_ch, w_ch,
                          asrc_v, adst_v, tmp_v)
                return carry

            lax.fori_loop(0, NCH // 2, mchunk, 0)
            plsc.subcore_barrier()
            pltpu.sync_copy(msg_spm.at[reg].at[pl.ds(half * (NP // 2), NP // 2)],
                            msg_out.at[g].at[pl.ds(half * (NP // 2), NP // 2)])
            plsc.subcore_barrier()

    return k


def _gat_sc3():
    mesh = plsc.VectorSubcoreMesh(core_axis_name="c", subcore_axis_name="s",
                                  num_cores=2, num_subcores=16)

    @functools.partial(
        pl.kernel,
        out_type=jax.ShapeDtypeStruct((G // 4, NP, 4 * HID), F32),
        mesh=mesh,
        compiler_params=pltpu.CompilerParams(needs_layout_passes=False),
        scratch_types=[
            pltpu.VMEM_SHARED((4, NP, 4 * HID), F32),
            pltpu.VMEM((CHE,), I32),
            pltpu.VMEM((CHE,), I32),
            pltpu.VMEM((4, CHE), F32),
            pltpu.VMEM((4 * NP,), F32),
            pltpu.VMEM((4 * NP,), F32),
            pltpu.VMEM((CHE, 4 * HID), F32),
        ],
    )
    def k(xw_hbm, msgi_hbm, asrc_hbm, adst_hbm, sf_hbm, df_hbm, msg_out,
          msg_spm, s_ch, d_ch, w_ch, asrc_v, adst_v, tmp_v):
        cid = lax.axis_index("c")
        sid = lax.axis_index("s")
        lq = sid // 4
        part = sid % 4
        q = cid * 4 + lq
        pltpu.sync_copy(asrc_hbm.at[q], asrc_v)
        pltpu.sync_copy(adst_hbm.at[q], adst_v)

        @pl.when(part == 0)
        def _():
            pltpu.sync_copy(msgi_hbm.at[q], msg_spm.at[lq])

        plsc.subcore_barrier()

        def mchunk(j, carry):
            ch = part + 4 * j

            @pl.when(ch < NCH)
            def _():
                _do_chunk(ch, q, xw_hbm, sf_hbm, df_hbm,
                          msg_spm.at[lq], s_ch, d_ch, w_ch,
                          asrc_v, adst_v, tmp_v)
            return carry

        lax.fori_loop(0, (NCH + 3) // 4, mchunk, 0)
        plsc.subcore_barrier()

        @pl.when(part == 0)
        def _():
            pltpu.sync_copy(msg_spm.at[lq], msg_out.at[q])

    return k


def _tail_body(msg3_ref, den3_ref, b3_ref, pw_ref, mask_ref,
               Wih_f_ref, Whh_f_ref, bih_f_ref, bhh_f_ref,
               Wih_r_ref, Whh_r_ref, bih_r_ref, bhh_r_ref,
               Wmu_ref, bmu_ref, Wlv_ref, blv_ref, Wpi_ref, bpi_ref,
               mu_ref, lv_ref, pi_ref):
    pw = pw_ref[...][:, 0]
    valid = lax.broadcasted_iota(I32, (1, NP), 1) < N
    pooled_parts = []
    for r in range(4):
        m = msg3_ref[...][:, :, r * HID:(r + 1) * HID]
        dn = den3_ref[...][:, r, :]
        h3 = jnp.maximum(m / dn[..., None] + b3_ref[...][None, None, :], 0.0)
        logits = jnp.sum(h3 * pw[None, None, :], axis=-1)
        ex = jnp.where(valid, jnp.exp(logits), 0.0)
        denp = jnp.sum(ex, axis=-1)
        pooled_parts.append(
            jnp.sum(ex[..., None] * h3, axis=1) / (denp[:, None] + 1e-16))
    pooled = jnp.stack(pooled_parts, axis=1).reshape(G, HID)
    mask = mask_ref[...]
    ge = pooled.reshape(B, T, HID) * mask
    lengths = jnp.clip(jnp.sum(mask[:, :, 0], axis=1), 1, None).astype(I32)

    def lstm(Wih, Whh, bih, bhh, reverse):
        h = jnp.zeros((B, RNN), F32)
        c = jnp.zeros((B, RNN), F32)
        for kk in range(T):
            t = T - 1 - kk if reverse else kk
            xt = ge[:, t, :]
            g = (lax.dot_general(xt, Wih, (((1,), (1,)), ((), ())),
                                 precision=PREC) + bih[None, :] +
                 lax.dot_general(h, Whh, (((1,), (1,)), ((), ())),
                                 precision=PREC) + bhh[None, :])
            i, f, gg, o = jnp.split(g, 4, axis=-1)
            i = jax.nn.sigmoid(i)
            f = jax.nn.sigmoid(f)
            gg = jnp.tanh(gg)
            o = jax.nn.sigmoid(o)
            cn = f * c + i * gg
            hn = o * jnp.tanh(cn)
            ok = (t < lengths)[:, None]
            h = jnp.where(ok, hn, h)
            c = jnp.where(ok, cn, c)
        return h

    hf = lstm(Wih_f_ref[...], Whh_f_ref[...], bih_f_ref[...], bhh_f_ref[...],
              False)
    hr = lstm(Wih_r_ref[...], Whh_r_ref[...], bih_r_ref[...], bhh_r_ref[...],
              True)
    feat = jnp.concatenate([hf, hr], axis=1)
    mu_ref[...] = lax.dot_general(feat, Wmu_ref[...], (((1,), (1,)), ((), ())),
                                  precision=PREC) + bmu_ref[...][None, :]
    lv_ref[...] = lax.dot_general(feat, Wlv_ref[...], (((1,), (1,)), ((), ())),
                                  precision=PREC) + blv_ref[...][None, :]
    pi_ref[...] = lax.dot_general(feat, Wpi_ref[...], (((1,), (1,)), ((), ())),
                                  precision=PREC) + bpi_ref[...][None, :]


def _tail(msg3, den3, b3, pool_W, mask, Wih_f, Whh_f, bih_f, bhh_f,
          Wih_r, Whh_r, bih_r, bhh_r, Wmu, bmu, Wlv, blv, Wpi, bpi):
    return pl.pallas_call(
        _tail_body,
        out_shape=(
            jax.ShapeDtypeStruct((B, K * LAT), F32),
            jax.ShapeDtypeStruct((B, K * LAT), F32),
            jax.ShapeDtypeStruct((B, K), F32),
        ),
    )(msg3, den3, b3, pool_W, mask, Wih_f, Whh_f, bih_f, bhh_f,
      Wih_r, Whh_r, bih_r, bhh_r, Wmu, bmu, Wlv, blv, Wpi, bpi)


def kernel(x, edge_index, mask, W1, as1, ad1, b1, W2, as2, ad2, b2, W3, as3,
           ad3, b3, pool_W, pool_b, Wih_f, Whh_f, bih_f, bhh_f, Wih_r, Whh_r,
           bih_r, bhh_r, Wmu, bmu, Wlv, blv, Wpi, bpi):
    del pool_b
    x_p = jnp.pad(x.reshape(G, N, FD), ((0, 0), (0, NP - N), (0, 0)))
    x_p = x_p.reshape(G, 1, NP, FD)
    s_flat = edge_index[0]
    d_flat = edge_index[1]
    s3 = s_flat.reshape(ENCH, 1, ECH)
    d3 = d_flat.reshape(ENCH, 1, ECH)

    gat12 = _gat_sc12()
    gat3 = _gat_sc3()
    P = G * HEADS

    xw, asr, ads, mi, di = _prep1(x_p, W1, as1, ad1, HEADS)
    den1 = _edgew(s3, d3, asr.reshape(P, NP), ads.reshape(P, NP),
                  di.reshape(P, NP), P)
    msg1 = gat12(xw, mi, asr.reshape(G, 4 * NP), ads.reshape(G, 4 * NP),
                 s_flat, d_flat)
    xw, asr, ads, mi, di = _prepL(msg1, den1.reshape(G, HEADS, NP),
                                  b1.reshape(HEADS, HID), W2, as2, ad2,
                                  HEADS, HEADS)
    den2 = _edgew(s3, d3, asr.reshape(P, NP), ads.reshape(P, NP),
                  di.reshape(P, NP), P)
    msg2 = gat12(xw, mi, asr.reshape(G, 4 * NP), ads.reshape(G, 4 * NP),
                 s_flat, d_flat)
    xw, asr, ads, mi, di = _prep3(msg2, den2.reshape(G, HEADS, NP),
                                  b2.reshape(HEADS, HID), W3, as3, ad3,
                                  HEADS)
    den3 = _edgew(s3, d3, asr.reshape(G, NP), ads.reshape(G, NP),
                  di.reshape(G, NP), G)
    msg3 = gat3(xw, mi, asr.reshape(G // 4, 4 * NP),
                ads.reshape(G // 4, 4 * NP), s_flat, d_flat)

    mu, lv, pi = _tail(msg3, den3.reshape(G // 4, 4, NP), b3, pool_W, mask,
                       Wih_f, Whh_f, bih_f, bhh_f, Wih_r, Whh_r, bih_r, bhh_r,
                       Wmu, bmu, Wlv, blv, Wpi, bpi)
    return mu.reshape(B, K, LAT), lv.reshape(B, K, LAT), pi


# speedup vs baseline: 111.6433x; 1.1031x over previous
import functools

import jax
import jax.numpy as jnp
from jax import lax
from jax.experimental import pallas as pl
from jax.experimental.pallas import tpu as pltpu
from jax.experimental.pallas import tpu_sc as plsc

F32 = jnp.float32
I32 = jnp.int32

B, T, N, FD = 4, 8, 1000, 128
HID, HEADS, RNN, LAT, K = 32, 4, 128, 64, 32
G = B * T
NP = 1008
NE = 16000
CHE = 80
EPP = NE // 4
CPP = EPP // CHE
NPR = CPP // 2
ECH = 640
ENCH = NE // ECH
PREC = None


def _prep_common(x, W_ref, as_ref, ad_ref, heads):
    xw = jnp.dot(x, W_ref[...], preferred_element_type=F32, precision=PREC)
    asrs, adss, wselfs, mparts = [], [], [], []
    for h in range(heads):
        xwh = xw[:, h * HID:(h + 1) * HID]
        asr = jnp.sum(xwh * as_ref[h][None, :], axis=-1)
        ads = jnp.sum(xwh * ad_ref[h][None, :], axis=-1)
        e = asr + ads
        w = jnp.exp(jnp.maximum(e, 0.2 * e))
        asrs.append(asr)
        adss.append(ads)
        wselfs.append(w)
        mparts.append(xwh * w[:, None])
    return xw, asrs, adss, wselfs, jnp.concatenate(mparts, axis=-1)


def _normalize(msg_ref, den_ref, bias_ref, r, heads_in):
    parts = []
    for hh in range(heads_in):
        m = msg_ref[r][:, hh * HID:(hh + 1) * HID]
        dn = den_ref[r, hh]
        parts.append(jnp.maximum(m / dn[:, None] + bias_ref[hh][None, :], 0.0))
    return jnp.concatenate(parts, axis=-1)


def _prep1_body(x_ref, W_ref, as_ref, ad_ref, xw_ref, asrc_ref, adst_ref,
                msgi_ref, deni_ref, *, heads):
    xw, asrs, adss, wselfs, msgi = _prep_common(
        x_ref[0, 0], W_ref, as_ref, ad_ref, heads)
    xw_ref[0] = xw
    msgi_ref[0] = msgi
    for h in range(heads):
        asrc_ref[h, 0] = asrs[h]
        adst_ref[h, 0] = adss[h]
        deni_ref[h, 0] = wselfs[h]


def _prepL_body(msg_ref, den_ref, bias_ref, W_ref, as_ref, ad_ref, xw_ref,
                asrc_ref, adst_ref, msgi_ref, deni_ref, *, heads, heads_in):
    x = _normalize(msg_ref, den_ref, bias_ref, 0, heads_in)
    xw, asrs, adss, wselfs, msgi = _prep_common(x, W_ref, as_ref, ad_ref,
                                                heads)
    xw_ref[0] = xw
    msgi_ref[0] = msgi
    for h in range(heads):
        asrc_ref[h, 0] = asrs[h]
        adst_ref[h, 0] = adss[h]
        deni_ref[h, 0] = wselfs[h]


def _prep3_body(msg_ref, den_ref, bias_ref, W_ref, as_ref, ad_ref, xw_ref,
                asrc_ref, adst_ref, msgi_ref, deni_ref, *, heads_in):
    for r in range(4):
        x = _normalize(msg_ref, den_ref, bias_ref, r, heads_in)
        xw, asrs, adss, wselfs, msgi = _prep_common(x, W_ref, as_ref, ad_ref,
                                                    1)
        xw_ref[0, :, r * HID:(r + 1) * HID] = xw
        msgi_ref[0, :, r * HID:(r + 1) * HID] = msgi
        asrc_ref[r, 0] = asrs[0]
        adst_ref[r, 0] = adss[0]
        deni_ref[r, 0] = wselfs[0]


def _prep_outs(heads):
    P = G * heads
    out_shapes = (
        jax.ShapeDtypeStruct((G, NP, heads * HID), F32),
        jax.ShapeDtypeStruct((P, 1, NP), F32),
        jax.ShapeDtypeStruct((P, 1, NP), F32),
        jax.ShapeDtypeStruct((G, NP, heads * HID), F32),
        jax.ShapeDtypeStruct((P, 1, NP), F32),
    )
    xw_spec = pl.BlockSpec((1, NP, heads * HID), lambda g: (g, 0, 0))
    pv_spec = pl.BlockSpec((heads, 1, NP), lambda g: (g, 0, 0))
    out_specs = (xw_spec, pv_spec, pv_spec, xw_spec, pv_spec)
    return out_shapes, out_specs


def _prep1(x_p, W, a_s, a_d, heads):
    out_shapes, out_specs = _prep_outs(heads)
    fin = W.shape[0]
    return pl.pallas_call(
        functools.partial(_prep1_body, heads=heads),
        grid=(G,),
        in_specs=[
            pl.BlockSpec((1, 1, NP, fin), lambda g: (g, 0, 0, 0)),
            pl.BlockSpec((fin, heads * HID), lambda g: (0, 0)),
            pl.BlockSpec((heads, HID), lambda g: (0, 0)),
            pl.BlockSpec((heads, HID), lambda g: (0, 0)),
        ],
        out_specs=out_specs,
        out_shape=out_shapes,
    )(x_p, W, a_s, a_d)


def _prepL(msg, den, bias, W, a_s, a_d, heads, heads_in):
    out_shapes, out_specs = _prep_outs(heads)
    fin = W.shape[0]
    return pl.pallas_call(
        functools.partial(_prepL_body, heads=heads, heads_in=heads_in),
        grid=(G,),
        in_specs=[
            pl.BlockSpec((1, NP, heads_in * HID), lambda g: (g, 0, 0)),
            pl.BlockSpec((1, heads_in, NP), lambda g: (g, 0, 0)),
            pl.BlockSpec((heads_in, HID), lambda g: (0, 0)),
            pl.BlockSpec((fin, heads * HID), lambda g: (0, 0)),
            pl.BlockSpec((heads, HID), lambda g: (0, 0)),
            pl.BlockSpec((heads, HID), lambda g: (0, 0)),
        ],
        out_specs=out_specs,
        out_shape=out_shapes,
    )(msg, den, bias, W, a_s, a_d)


def _prep3(msg, den, bias, W, a_s, a_d, heads_in):
    out_shapes = (
        jax.ShapeDtypeStruct((G // 4, NP, 4 * HID), F32),
        jax.ShapeDtypeStruct((G, 1, NP), F32),
        jax.ShapeDtypeStruct((G, 1, NP), F32),
        jax.ShapeDtypeStruct((G // 4, NP, 4 * HID), F32),
        jax.ShapeDtypeStruct((G, 1, NP), F32),
    )
    xw_spec = pl.BlockSpec((1, NP, 4 * HID), lambda q: (q, 0, 0))
    pv_spec = pl.BlockSpec((4, 1, NP), lambda q: (q, 0, 0))
    out_specs = (xw_spec, pv_spec, pv_spec, xw_spec, pv_spec)
    fin = W.shape[0]
    return pl.pallas_call(
        functools.partial(_prep3_body, heads_in=heads_in),
        grid=(G // 4,),
        in_specs=[
            pl.BlockSpec((4, NP, heads_in * HID), lambda q: (q, 0, 0)),
            pl.BlockSpec((4, heads_in, NP), lambda q: (q, 0, 0)),
            pl.BlockSpec((heads_in, HID), lambda q: (0, 0)),
            pl.BlockSpec((fin, HID), lambda q: (0, 0)),
            pl.BlockSpec((1, HID), lambda q: (0, 0)),
            pl.BlockSpec((1, HID), lambda q: (0, 0)),
        ],
        out_specs=out_specs,
        out_shape=out_shapes,
    )(msg, den, bias, W, a_s, a_d)


def _edgew_body(s_ref, d_ref, asrc_ref, adst_ref, deni_ref, den_ref):
    ch = pl.program_id(0)
    sv = s_ref[0, 0]
    dv = d_ref[0, 0]
    nodes = lax.broadcasted_iota(I32, (ECH, NP), 1)
    oh_s = (sv[:, None] == nodes).astype(F32)
    oh_d = (dv[:, None] == nodes).astype(F32)
    asrc_e = lax.dot_general(asrc_ref[...], oh_s, (((1,), (1,)), ((), ())),
                             preferred_element_type=F32, precision=PREC)
    adst_e = lax.dot_general(adst_ref[...], oh_d, (((1,), (1,)), ((), ())),
                             preferred_element_type=F32, precision=PREC)
    e = asrc_e + adst_e
    w = jnp.exp(jnp.maximum(e, 0.2 * e))

    @pl.when(ch == 0)
    def _():
        den_ref[...] = deni_ref[...]

    den_ref[...] += lax.dot_general(w, oh_d, (((1,), (0,)), ((), ())),
                                    preferred_element_type=F32, precision=PREC)


def _edgew(s3, d3, asrc, adst, deni, P):
    return pl.pallas_call(
        _edgew_body,
        grid=(ENCH,),
        in_specs=[
            pl.BlockSpec((1, 1, ECH), lambda ch: (ch, 0, 0)),
            pl.BlockSpec((1, 1, ECH), lambda ch: (ch, 0, 0)),
            pl.BlockSpec((P, NP), lambda ch: (0, 0)),
            pl.BlockSpec((P, NP), lambda ch: (0, 0)),
            pl.BlockSpec((P, NP), lambda ch: (0, 0)),
        ],
        out_specs=pl.BlockSpec((P, NP), lambda ch: (0, 0)),
        out_shape=jax.ShapeDtypeStruct((P, NP), F32),
    )(s3, d3, asrc, adst, deni)


def _scale_rows(tmp_v, w_ch):
    for j in range(CHE // 16):
        base = j * 16
        wvecs = [w_ch[h, pl.ds(base, 16)] for h in range(4)]
        for lane in range(16):
            e = base + lane
            for h in range(4):
                ws = wvecs[h][lane]
                tmp_v[e, pl.ds(h * 32, 16)] = tmp_v[e, pl.ds(h * 32, 16)] * ws
                tmp_v[e, pl.ds(h * 32 + 16, 16)] = (
                    tmp_v[e, pl.ds(h * 32 + 16, 16)] * ws)


def _wgrp(k, s_v, d_v, w_ch, asrc_v, adst_v):
    def grp(kk, c2):
        sv = s_v[pl.ds(k * CHE + kk * 16, 16)]
        dv = d_v[pl.ds(k * CHE + kk * 16, 16)]
        for h in range(4):
            a1 = plsc.load_gather(asrc_v, [sv + h * NP])
            a2 = plsc.load_gather(adst_v, [dv + h * NP])
            e = a1 + a2
            w_ch[h, pl.ds(kk * 16, 16)] = jnp.exp(jnp.maximum(e, 0.2 * e))
        return c2

    lax.fori_loop(0, CHE // 16, grp, 0)


def _gat_sc(NOBJ):
    NPASS = NOBJ // 8
    mesh = plsc.VectorSubcoreMesh(core_axis_name="c", subcore_axis_name="s",
                                  num_cores=2, num_subcores=16)

    @functools.partial(
        pl.kernel,
        out_type=jax.ShapeDtypeStruct((NOBJ, NP, 4 * HID), F32),
        mesh=mesh,
        compiler_params=pltpu.CompilerParams(needs_layout_passes=False),
        scratch_types=[
            pltpu.VMEM_SHARED((4, NP, 4 * HID), F32),
            pltpu.VMEM((EPP,), I32),
            pltpu.VMEM((EPP,), I32),
            pltpu.VMEM((4, CHE), F32),
            pltpu.VMEM((4, CHE), F32),
            pltpu.VMEM((4 * NP,), F32),
            pltpu.VMEM((4 * NP,), F32),
            pltpu.VMEM((CHE, 4 * HID), F32),
            pltpu.VMEM((CHE, 4 * HID), F32),
            pltpu.SemaphoreType.DMA,
            pltpu.SemaphoreType.DMA,
        ],
    )
    def k(xw_hbm, msgi_hbm, asrc_hbm, adst_hbm, sf_hbm, df_hbm, msg_out,
          msg_spm, s_v, d_v, w_a, w_b, asrc_v, adst_v, tmp_a, tmp_b,
          sem_a, sem_b):
        cid = lax.axis_index("c")
        sid = lax.axis_index("s")
        reg = sid // 4
        part = sid % 4
        pltpu.sync_copy(sf_hbm.at[pl.ds(part * EPP, EPP)], s_v)
        pltpu.sync_copy(df_hbm.at[pl.ds(part * EPP, EPP)], d_v)

        def gstart(obj, k_local, tmp, sem):
            idx = s_v.at[pl.ds(k_local * CHE, CHE)]
            pltpu.make_async_copy(xw_hbm.at[obj].at[idx], tmp, sem).start()

        def gwait(obj, k_local, tmp, sem):
            idx = s_v.at[pl.ds(k_local * CHE, CHE)]
            pltpu.make_async_copy(xw_hbm.at[obj].at[idx], tmp, sem).wait()

        def process(obj, k_local, tmp, w_ch, sem):
            _wgrp(k_local, s_v, d_v, w_ch, asrc_v, adst_v)
            gwait(obj, k_local, tmp, sem)
            _scale_rows(tmp, w_ch)
            idx = d_v.at[pl.ds(k_local * CHE, CHE)]
            pltpu.sync_copy(tmp, msg_spm.at[reg].at[idx], add=True)

        for pp in range(NPASS):
            obj = pp * 8 + cid * 4 + reg
            pltpu.sync_copy(asrc_hbm.at[obj], asrc_v)
            pltpu.sync_copy(adst_hbm.at[obj], adst_v)

            @pl.when(part == 0)
            def _():
                pltpu.sync_copy(msgi_hbm.at[obj], msg_spm.at[reg])

            plsc.subcore_barrier()
            gstart(obj, 0, tmp_a, sem_a)

            def pair(jj, carry):
                c0 = 2 * jj
                gstart(obj, c0 + 1, tmp_b, sem_b)
                process(obj, c0, tmp_a, w_a, sem_a)

                @pl.when(jj + 1 < NPR)
                def _():
                    gstart(obj, c0 + 2, tmp_a, sem_a)

                process(obj, c0 + 1, tmp_b, w_b, sem_b)
                return carry

            lax.fori_loop(0, NPR, pair, 0)
            plsc.subcore_barrier()

            @pl.when(part < 3)
            def _():
                pltpu.sync_copy(msg_spm.at[reg].at[pl.ds(part * 256, 256)],
                                msg_out.at[obj].at[pl.ds(part * 256, 256)])

            @pl.when(part == 3)
            def _():
                pltpu.sync_copy(msg_spm.at[reg].at[pl.ds(768, NP - 768)],
                                msg_out.at[obj].at[pl.ds(768, NP - 768)])

            plsc.subcore_barrier()

    return k


def _tail_body(msg3_ref, den3_ref, b3_ref, pw_ref, mask_ref,
               Wih_f_ref, Whh_f_ref, bih_f_ref, bhh_f_ref,
               Wih_r_ref, Whh_r_ref, bih_r_ref, bhh_r_ref,
               Wmu_ref, bmu_ref, Wlv_ref, blv_ref, Wpi_ref, bpi_ref,
               mu_ref, lv_ref, pi_ref):
    pw = pw_ref[...][:, 0]
    valid = lax.broadcasted_iota(I32, (1, NP), 1) < N
    pooled_parts = []
    for r in range(4):
        m = msg3_ref[...][:, :, r * HID:(r + 1) * HID]
        dn = den3_ref[...][:, r, :]
        h3 = jnp.maximum(m / dn[..., None] + b3_ref[...][None, None, :], 0.0)
        logits = jnp.sum(h3 * pw[None, None, :], axis=-1)
        ex = jnp.where(valid, jnp.exp(logits), 0.0)
        denp = jnp.sum(ex, axis=-1)
        pooled_parts.append(
            jnp.sum(ex[..., None] * h3, axis=1) / (denp[:, None] + 1e-16))
    pooled = jnp.stack(pooled_parts, axis=1).reshape(G, HID)
    mask = mask_ref[...]
    ge = pooled.reshape(B, T, HID) * mask
    lengths = jnp.clip(jnp.sum(mask[:, :, 0], axis=1), 1, None).astype(I32)

    def lstm(Wih, Whh, bih, bhh, reverse):
        h = jnp.zeros((B, RNN), F32)
        c = jnp.zeros((B, RNN), F32)
        for kk in range(T):
            t = T - 1 - kk if reverse else kk
            xt = ge[:, t, :]
            g = (lax.dot_general(xt, Wih, (((1,), (1,)), ((), ())),
                                 precision=PREC) + bih[None, :] +
                 lax.dot_general(h, Whh, (((1,), (1,)), ((), ())),
                                 precision=PREC) + bhh[None, :])
            i, f, gg, o = jnp.split(g, 4, axis=-1)
            i = jax.nn.sigmoid(i)
            f = jax.nn.sigmoid(f)
            gg = jnp.tanh(gg)
            o = jax.nn.sigmoid(o)
            cn = f * c + i * gg
            hn = o * jnp.tanh(cn)
            ok = (t < lengths)[:, None]
            h = jnp.where(ok, hn, h)
            c = jnp.where(ok, cn, c)
        return h

    hf = lstm(Wih_f_ref[...], Whh_f_ref[...], bih_f_ref[...], bhh_f_ref[...],
              False)
    hr = lstm(Wih_r_ref[...], Whh_r_ref[...], bih_r_ref[...], bhh_r_ref[...],
              True)
    feat = jnp.concatenate([hf, hr], axis=1)
    mu_ref[...] = lax.dot_general(feat, Wmu_ref[...], (((1,), (1,)), ((), ())),
                                  precision=PREC) + bmu_ref[...][None, :]
    lv_ref[...] = lax.dot_general(feat, Wlv_ref[...], (((1,), (1,)), ((), ())),
                                  precision=PREC) + blv_ref[...][None, :]
    pi_ref[...] = lax.dot_general(feat, Wpi_ref[...], (((1,), (1,)), ((), ())),
                                  precision=PREC) + bpi_ref[...][None, :]


def _tail(msg3, den3, b3, pool_W, mask, Wih_f, Whh_f, bih_f, bhh_f,
          Wih_r, Whh_r, bih_r, bhh_r, Wmu, bmu, Wlv, blv, Wpi, bpi):
    return pl.pallas_call(
        _tail_body,
        out_shape=(
            jax.ShapeDtypeStruct((B, K * LAT), F32),
            jax.ShapeDtypeStruct((B, K * LAT), F32),
            jax.ShapeDtypeStruct((B, K), F32),
        ),
    )(msg3, den3, b3, pool_W, mask, Wih_f, Whh_f, bih_f, bhh_f,
      Wih_r, Whh_r, bih_r, bhh_r, Wmu, bmu, Wlv, blv, Wpi, bpi)


def kernel(x, edge_index, mask, W1, as1, ad1, b1, W2, as2, ad2, b2, W3, as3,
           ad3, b3, pool_W, pool_b, Wih_f, Whh_f, bih_f, bhh_f, Wih_r, Whh_r,
           bih_r, bhh_r, Wmu, bmu, Wlv, blv, Wpi, bpi):
    del pool_b
    x_p = jnp.pad(x.reshape(G, N, FD), ((0, 0), (0, NP - N), (0, 0)))
    x_p = x_p.reshape(G, 1, NP, FD)
    s_flat = edge_index[0]
    d_flat = edge_index[1]
    s3 = s_flat.reshape(ENCH, 1, ECH)
    d3 = d_flat.reshape(ENCH, 1, ECH)

    gat12 = _gat_sc(G)
    gat3 = _gat_sc(G // 4)
    P = G * HEADS

    xw, asr, ads, mi, di = _prep1(x_p, W1, as1, ad1, HEADS)
    den1 = _edgew(s3, d3, asr.reshape(P, NP), ads.reshape(P, NP),
                  di.reshape(P, NP), P)
    msg1 = gat12(xw, mi, asr.reshape(G, 4 * NP), ads.reshape(G, 4 * NP),
                 s_flat, d_flat)
    xw, asr, ads, mi, di = _prepL(msg1, den1.reshape(G, HEADS, NP),
                                  b1.reshape(HEADS, HID), W2, as2, ad2,
                                  HEADS, HEADS)
    den2 = _edgew(s3, d3, asr.reshape(P, NP), ads.reshape(P, NP),
                  di.reshape(P, NP), P)
    msg2 = gat12(xw, mi, asr.reshape(G, 4 * NP), ads.reshape(G, 4 * NP),
                 s_flat, d_flat)
    xw, asr, ads, mi, di = _prep3(msg2, den2.reshape(G, HEADS, NP),
                                  b2.reshape(HEADS, HID), W3, as3, ad3,
                                  HEADS)
    den3 = _edgew(s3, d3, asr.reshape(G, NP), ads.reshape(G, NP),
                  di.reshape(G, NP), G)
    msg3 = gat3(xw, mi, asr.reshape(G // 4, 4 * NP),
                ads.reshape(G // 4, 4 * NP), s_flat, d_flat)

    mu, lv, pi = _tail(msg3, den3.reshape(G // 4, 4, NP), b3, pool_W, mask,
                       Wih_f, Whh_f, bih_f, bhh_f, Wih_r, Whh_r, bih_r, bhh_r,
                       Wmu, bmu, Wlv, blv, Wpi, bpi)
    return mu.reshape(B, K, LAT), lv.reshape(B, K, LAT), pi


# speedup vs baseline: 137.6531x; 1.2330x over previous
import functools

import jax
import jax.numpy as jnp
from jax import lax
from jax.experimental import pallas as pl
from jax.experimental.pallas import tpu as pltpu
from jax.experimental.pallas import tpu_sc as plsc

F32 = jnp.float32
I32 = jnp.int32

B, T, N, FD = 4, 8, 1000, 128
HID, HEADS, RNN, LAT, K = 32, 4, 128, 64, 32
G = B * T
NP = 1008
NE = 16000
CHE = 80
EPP = NE // 4
CPP = EPP // CHE
NPR = CPP // 2
ECH = 640
ENCH = NE // ECH
PREC = None


def _head_sel(heads, fout):
    return (lax.broadcasted_iota(I32, (heads, fout), 1) // HID ==
            lax.broadcasted_iota(I32, (heads, fout), 0)).astype(F32)


def _prep_common(x, W_ref, as_ref, ad_ref, heads):
    fout = heads * HID
    xw = jnp.dot(x, W_ref[...], preferred_element_type=F32, precision=PREC)
    as_cat = jnp.concatenate([as_ref[h] for h in range(heads)])
    ad_cat = jnp.concatenate([ad_ref[h] for h in range(heads)])
    rows = lax.broadcasted_iota(I32, (fout, 2 * heads), 0) // HID
    cols = lax.broadcasted_iota(I32, (fout, 2 * heads), 1)
    pick = jnp.where(cols < heads, as_cat[:, None], ad_cat[:, None])
    A = jnp.where(rows == jnp.where(cols < heads, cols, cols - heads),
                  pick, 0.0)
    aa = jnp.dot(xw, A, preferred_element_type=F32, precision=PREC)
    e = aa[:, :heads] + aa[:, heads:]
    wself = jnp.exp(jnp.maximum(e, 0.2 * e))
    wexp = jnp.dot(wself, _head_sel(heads, fout),
                   preferred_element_type=F32, precision=PREC)
    msgi = xw * wexp
    aux = jnp.concatenate([aa, wself], axis=1)
    return xw, aux, msgi


def _normalize(msg, dn, bias_ref, heads_in):
    dn_exp = jnp.dot(dn, _head_sel(heads_in, heads_in * HID),
                     preferred_element_type=F32, precision=PREC)
    return jnp.maximum(msg / dn_exp + bias_ref[0][None, :], 0.0)


def _prep1_body(x_ref, W_ref, as_ref, ad_ref, xw_ref, aux_ref, msgi_ref, *,
                heads):
    xw, aux, msgi = _prep_common(x_ref[0, 0], W_ref, as_ref, ad_ref, heads)
    xw_ref[0] = xw
    aux_ref[0] = aux
    msgi_ref[0] = msgi


def _prepL_body(msg_ref, den_ref, bias_ref, W_ref, as_ref, ad_ref, xw_ref,
                aux_ref, msgi_ref, *, heads, heads_in):
    x = _normalize(msg_ref[0], den_ref[0], bias_ref, heads_in)
    xw, aux, msgi = _prep_common(x, W_ref, as_ref, ad_ref, heads)
    xw_ref[0] = xw
    aux_ref[0] = aux
    msgi_ref[0] = msgi


def _prep3_body(msg_ref, den_ref, bias_ref, W_ref, as_ref, ad_ref, xw_ref,
                aux_ref, msgi_ref, *, heads_in):
    for r in range(4):
        x = _normalize(msg_ref[r], den_ref[r], bias_ref, heads_in)
        xw, aux, msgi = _prep_common(x, W_ref, as_ref, ad_ref, 1)
        xw_ref[0, :, r * HID:(r + 1) * HID] = xw
        msgi_ref[0, :, r * HID:(r + 1) * HID] = msgi
        aux_ref[r] = aux


def _prep1(x_p, W, a_s, a_d, heads):
    fin = W.shape[0]
    return pl.pallas_call(
        functools.partial(_prep1_body, heads=heads),
        grid=(G,),
        in_specs=[
            pl.BlockSpec((1, 1, NP, fin), lambda g: (g, 0, 0, 0)),
            pl.BlockSpec((fin, heads * HID), lambda g: (0, 0)),
            pl.BlockSpec((heads, HID), lambda g: (0, 0)),
            pl.BlockSpec((heads, HID), lambda g: (0, 0)),
        ],
        out_specs=(
            pl.BlockSpec((1, NP, heads * HID), lambda g: (g, 0, 0)),
            pl.BlockSpec((1, NP, 3 * heads), lambda g: (g, 0, 0)),
            pl.BlockSpec((1, NP, heads * HID), lambda g: (g, 0, 0)),
        ),
        out_shape=(
            jax.ShapeDtypeStruct((G, NP, heads * HID), F32),
            jax.ShapeDtypeStruct((G, NP, 3 * heads), F32),
            jax.ShapeDtypeStruct((G, NP, heads * HID), F32),
        ),
    )(x_p, W, a_s, a_d)


def _prepL(msg, den, bias, W, a_s, a_d, heads, heads_in):
    fin = W.shape[0]
    return pl.pallas_call(
        functools.partial(_prepL_body, heads=heads, heads_in=heads_in),
        grid=(G,),
        in_specs=[
            pl.BlockSpec((1, NP, heads_in * HID), lambda g: (g, 0, 0)),
            pl.BlockSpec((1, NP, heads_in), lambda g: (g, 0, 0)),
            pl.BlockSpec((1, heads_in * HID), lambda g: (0, 0)),
            pl.BlockSpec((fin, heads * HID), lambda g: (0, 0)),
            pl.BlockSpec((heads, HID), lambda g: (0, 0)),
            pl.BlockSpec((heads, HID), lambda g: (0, 0)),
        ],
        out_specs=(
            pl.BlockSpec((1, NP, heads * HID), lambda g: (g, 0, 0)),
            pl.BlockSpec((1, NP, 3 * heads), lambda g: (g, 0, 0)),
            pl.BlockSpec((1, NP, heads * HID), lambda g: (g, 0, 0)),
        ),
        out_shape=(
            jax.ShapeDtypeStruct((G, NP, heads * HID), F32),
            jax.ShapeDtypeStruct((G, NP, 3 * heads), F32),
            jax.ShapeDtypeStruct((G, NP, heads * HID), F32),
        ),
    )(msg, den, bias, W, a_s, a_d)


def _prep3(msg, den, bias, W, a_s, a_d, heads_in):
    fin = W.shape[0]
    return pl.pallas_call(
        functools.partial(_prep3_body, heads_in=heads_in),
        grid=(G // 4,),
        in_specs=[
            pl.BlockSpec((4, NP, heads_in * HID), lambda q: (q, 0, 0)),
            pl.BlockSpec((4, NP, heads_in), lambda q: (q, 0, 0)),
            pl.BlockSpec((1, heads_in * HID), lambda q: (0, 0)),
            pl.BlockSpec((fin, HID), lambda q: (0, 0)),
            pl.BlockSpec((1, HID), lambda q: (0, 0)),
            pl.BlockSpec((1, HID), lambda q: (0, 0)),
        ],
        out_specs=(
            pl.BlockSpec((1, NP, 4 * HID), lambda q: (q, 0, 0)),
            pl.BlockSpec((4, NP, 3), lambda q: (q, 0, 0)),
            pl.BlockSpec((1, NP, 4 * HID), lambda q: (q, 0, 0)),
        ),
        out_shape=(
            jax.ShapeDtypeStruct((G // 4, NP, 4 * HID), F32),
            jax.ShapeDtypeStruct((G, NP, 3), F32),
            jax.ShapeDtypeStruct((G // 4, NP, 4 * HID), F32),
        ),
    )(msg, den, bias, W, a_s, a_d)


def _edgew_body(s_ref, d_ref, asrc_ref, adst_ref, deni_ref, den_ref):
    ch = pl.program_id(0)
    sv = s_ref[0, 0]
    dv = d_ref[0, 0]
    nodes = lax.broadcasted_iota(I32, (ECH, NP), 1)
    oh_s = (sv[:, None] == nodes).astype(F32)
    oh_d = (dv[:, None] == nodes).astype(F32)
    asrc_e = lax.dot_general(asrc_ref[...], oh_s, (((1,), (1,)), ((), ())),
                             preferred_element_type=F32, precision=PREC)
    adst_e = lax.dot_general(adst_ref[...], oh_d, (((1,), (1,)), ((), ())),
                             preferred_element_type=F32, precision=PREC)
    e = asrc_e + adst_e
    w = jnp.exp(jnp.maximum(e, 0.2 * e))

    @pl.when(ch == 0)
    def _():
        den_ref[...] = deni_ref[...]

    den_ref[...] += lax.dot_general(w, oh_d, (((1,), (0,)), ((), ())),
                                    preferred_element_type=F32, precision=PREC)


def _edgew(s3, d3, asrc, adst, deni, P):
    return pl.pallas_call(
        _edgew_body,
        grid=(ENCH,),
        in_specs=[
            pl.BlockSpec((1, 1, ECH), lambda ch: (ch, 0, 0)),
            pl.BlockSpec((1, 1, ECH), lambda ch: (ch, 0, 0)),
            pl.BlockSpec((P, NP), lambda ch: (0, 0)),
            pl.BlockSpec((P, NP), lambda ch: (0, 0)),
            pl.BlockSpec((P, NP), lambda ch: (0, 0)),
        ],
        out_specs=pl.BlockSpec((P, NP), lambda ch: (0, 0)),
        out_shape=jax.ShapeDtypeStruct((P, NP), F32),
    )(s3, d3, asrc, adst, deni)


def _scale_rows(tmp_v, w_ch):
    for j in range(CHE // 16):
        base = j * 16
        wvecs = [w_ch[h, pl.ds(base, 16)] for h in range(4)]
        for lane in range(16):
            e = base + lane
            for h in range(4):
                ws = wvecs[h][lane]
                tmp_v[e, pl.ds(h * 32, 16)] = tmp_v[e, pl.ds(h * 32, 16)] * ws
                tmp_v[e, pl.ds(h * 32 + 16, 16)] = (
                    tmp_v[e, pl.ds(h * 32 + 16, 16)] * ws)


def _wgrp(k, s_v, d_v, w_ch, asrc_v, adst_v):
    def grp(kk, c2):
        sv = s_v[pl.ds(k * CHE + kk * 16, 16)]
        dv = d_v[pl.ds(k * CHE + kk * 16, 16)]
        for h in range(4):
            a1 = plsc.load_gather(asrc_v, [sv + h * NP])
            a2 = plsc.load_gather(adst_v, [dv + h * NP])
            e = a1 + a2
            w_ch[h, pl.ds(kk * 16, 16)] = jnp.exp(jnp.maximum(e, 0.2 * e))
        return c2

    lax.fori_loop(0, CHE // 16, grp, 0)


def _gat_sc(NOBJ):
    NPASS = NOBJ // 8
    mesh = plsc.VectorSubcoreMesh(core_axis_name="c", subcore_axis_name="s",
                                  num_cores=2, num_subcores=16)

    @functools.partial(
        pl.kernel,
        out_type=jax.ShapeDtypeStruct((NOBJ, NP, 4 * HID), F32),
        mesh=mesh,
        compiler_params=pltpu.CompilerParams(needs_layout_passes=False),
        scratch_types=[
            pltpu.VMEM_SHARED((4, NP, 4 * HID), F32),
            pltpu.VMEM((EPP,), I32),
            pltpu.VMEM((EPP,), I32),
            pltpu.VMEM((4, CHE), F32),
            pltpu.VMEM((4, CHE), F32),
            pltpu.VMEM((4 * NP,), F32),
            pltpu.VMEM((4 * NP,), F32),
            pltpu.VMEM((CHE, 4 * HID), F32),
            pltpu.VMEM((CHE, 4 * HID), F32),
            pltpu.SemaphoreType.DMA,
            pltpu.SemaphoreType.DMA,
        ],
    )
    def k(xw_hbm, msgi_hbm, asrc_hbm, adst_hbm, sf_hbm, df_hbm, msg_out,
          msg_spm, s_v, d_v, w_a, w_b, asrc_v, adst_v, tmp_a, tmp_b,
          sem_a, sem_b):
        cid = lax.axis_index("c")
        sid = lax.axis_index("s")
        reg = sid // 4
        part = sid % 4
        pltpu.sync_copy(sf_hbm.at[pl.ds(part * EPP, EPP)], s_v)
        pltpu.sync_copy(df_hbm.at[pl.ds(part * EPP, EPP)], d_v)

        def gstart(obj, k_local, tmp, sem):
            idx = s_v.at[pl.ds(k_local * CHE, CHE)]
            pltpu.make_async_copy(xw_hbm.at[obj].at[idx], tmp, sem).start()

        def gwait(obj, k_local, tmp, sem):
            idx = s_v.at[pl.ds(k_local * CHE, CHE)]
            pltpu.make_async_copy(xw_hbm.at[obj].at[idx], tmp, sem).wait()

        def process(obj, k_local, tmp, w_ch, sem):
            _wgrp(k_local, s_v, d_v, w_ch, asrc_v, adst_v)
            gwait(obj, k_local, tmp, sem)
            _scale_rows(tmp, w_ch)
            idx = d_v.at[pl.ds(k_local * CHE, CHE)]
            pltpu.sync_copy(tmp, msg_spm.at[reg].at[idx], add=True)

        for pp in range(NPASS):
            obj = pp * 8 + cid * 4 + reg
            pltpu.sync_copy(asrc_hbm.at[obj], asrc_v)
            pltpu.sync_copy(adst_hbm.at[obj], adst_v)

            @pl.when(part == 0)
            def _():
                pltpu.sync_copy(msgi_hbm.at[obj], msg_spm.at[reg])

            plsc.subcore_barrier()
            gstart(obj, 0, tmp_a, sem_a)

            def pair(jj, carry):
                c0 = 2 * jj
                gstart(obj, c0 + 1, tmp_b, sem_b)
                process(obj, c0, tmp_a, w_a, sem_a)

                @pl.when(jj + 1 < NPR)
                def _():
                    gstart(obj, c0 + 2, tmp_a, sem_a)

                process(obj, c0 + 1, tmp_b, w_b, sem_b)
                return carry

            lax.fori_loop(0, NPR, pair, 0)
            plsc.subcore_barrier()

            @pl.when(part < 3)
            def _():
                pltpu.sync_copy(msg_spm.at[reg].at[pl.ds(part * 256, 256)],
                                msg_out.at[obj].at[pl.ds(part * 256, 256)])

            @pl.when(part == 3)
            def _():
                pltpu.sync_copy(msg_spm.at[reg].at[pl.ds(768, NP - 768)],
                                msg_out.at[obj].at[pl.ds(768, NP - 768)])

            plsc.subcore_barrier()

    return k


def _tail_body(msg3_ref, den3_ref, b3_ref, pw_ref, mask_ref,
               Wih_f_ref, Whh_f_ref, bih_f_ref, bhh_f_ref,
               Wih_r_ref, Whh_r_ref, bih_r_ref, bhh_r_ref,
               Wmu_ref, bmu_ref, Wlv_ref, blv_ref, Wpi_ref, bpi_ref,
               mu_ref, lv_ref, pi_ref):
    pw = pw_ref[...][:, 0]
    valid = lax.broadcasted_iota(I32, (1, NP), 1) < N
    pooled_parts = []
    for r in range(4):
        m = msg3_ref[...][:, :, r * HID:(r + 1) * HID]
        dn = den3_ref[...][:, r, :]
        h3 = jnp.maximum(m / dn[..., None] + b3_ref[...][None, None, :], 0.0)
        logits = jnp.sum(h3 * pw[None, None, :], axis=-1)
        ex = jnp.where(valid, jnp.exp(logits), 0.0)
        denp = jnp.sum(ex, axis=-1)
        pooled_parts.append(
            jnp.sum(ex[..., None] * h3, axis=1) / (denp[:, None] + 1e-16))
    pooled = jnp.stack(pooled_parts, axis=1).reshape(G, HID)
    mask = mask_ref[...]
    ge = pooled.reshape(B, T, HID) * mask
    lengths = jnp.clip(jnp.sum(mask[:, :, 0], axis=1), 1, None).astype(I32)

    def lstm(Wih, Whh, bih, bhh, reverse):
        h = jnp.zeros((B, RNN), F32)
        c = jnp.zeros((B, RNN), F32)
        for kk in range(T):
            t = T - 1 - kk if reverse else kk
            xt = ge[:, t, :]
            g = (lax.dot_general(xt, Wih, (((1,), (1,)), ((), ())),
                                 precision=PREC) + bih[None, :] +
                 lax.dot_general(h, Whh, (((1,), (1,)), ((), ())),
                                 precision=PREC) + bhh[None, :])
            i, f, gg, o = jnp.split(g, 4, axis=-1)
            i = jax.nn.sigmoid(i)
            f = jax.nn.sigmoid(f)
            gg = jnp.tanh(gg)
            o = jax.nn.sigmoid(o)
            cn = f * c + i * gg
            hn = o * jnp.tanh(cn)
            ok = (t < lengths)[:, None]
            h = jnp.where(ok, hn, h)
            c = jnp.where(ok, cn, c)
        return h

    hf = lstm(Wih_f_ref[...], Whh_f_ref[...], bih_f_ref[...], bhh_f_ref[...],
              False)
    hr = lstm(Wih_r_ref[...], Whh_r_ref[...], bih_r_ref[...], bhh_r_ref[...],
              True)
    feat = jnp.concatenate([hf, hr], axis=1)
    mu_ref[...] = lax.dot_general(feat, Wmu_ref[...], (((1,), (1,)), ((), ())),
                                  precision=PREC) + bmu_ref[...][None, :]
    lv_ref[...] = lax.dot_general(feat, Wlv_ref[...], (((1,), (1,)), ((), ())),
                                  precision=PREC) + blv_ref[...][None, :]
    pi_ref[...] = lax.dot_general(feat, Wpi_ref[...], (((1,), (1,)), ((), ())),
                                  precision=PREC) + bpi_ref[...][None, :]


def _tail(msg3, den3, b3, pool_W, mask, Wih_f, Whh_f, bih_f, bhh_f,
          Wih_r, Whh_r, bih_r, bhh_r, Wmu, bmu, Wlv, blv, Wpi, bpi):
    return pl.pallas_call(
        _tail_body,
        out_shape=(
            jax.ShapeDtypeStruct((B, K * LAT), F32),
            jax.ShapeDtypeStruct((B, K * LAT), F32),
            jax.ShapeDtypeStruct((B, K), F32),
        ),
    )(msg3, den3, b3, pool_W, mask, Wih_f, Whh_f, bih_f, bhh_f,
      Wih_r, Whh_r, bih_r, bhh_r, Wmu, bmu, Wlv, blv, Wpi, bpi)


def kernel(x, edge_index, mask, W1, as1, ad1, b1, W2, as2, ad2, b2, W3, as3,
           ad3, b3, pool_W, pool_b, Wih_f, Whh_f, bih_f, bhh_f, Wih_r, Whh_r,
           bih_r, bhh_r, Wmu, bmu, Wlv, blv, Wpi, bpi):
    del pool_b
    x_p = jnp.pad(x.reshape(G, N, FD), ((0, 0), (0, NP - N), (0, 0)))
    x_p = x_p.reshape(G, 1, NP, FD)
    s_flat = edge_index[0]
    d_flat = edge_index[1]
    s3 = s_flat.reshape(ENCH, 1, ECH)
    d3 = d_flat.reshape(ENCH, 1, ECH)

    gat12 = _gat_sc(G)
    gat3 = _gat_sc(G // 4)
    P = G * HEADS

    def split_aux(aux, heads):
        auxt = jnp.transpose(aux, (0, 2, 1))
        return (auxt[:, :heads], auxt[:, heads:2 * heads],
                auxt[:, 2 * heads:])

    xw, aux, mi = _prep1(x_p, W1, as1, ad1, HEADS)
    asr, ads, di = split_aux(aux, HEADS)
    den1 = _edgew(s3, d3, asr.reshape(P, NP), ads.reshape(P, NP),
                  di.reshape(P, NP), P)
    msg1 = gat12(xw, mi, asr.reshape(G, 4 * NP), ads.reshape(G, 4 * NP),
                 s_flat, d_flat)
    den1g = jnp.transpose(den1.reshape(G, HEADS, NP), (0, 2, 1))
    xw, aux, mi = _prepL(msg1, den1g, b1.reshape(1, HEADS * HID), W2, as2,
                         ad2, HEADS, HEADS)
    asr, ads, di = split_aux(aux, HEADS)
    den2 = _edgew(s3, d3, asr.reshape(P, NP), ads.reshape(P, NP),
                  di.reshape(P, NP), P)
    msg2 = gat12(xw, mi, asr.reshape(G, 4 * NP), ads.reshape(G, 4 * NP),
                 s_flat, d_flat)
    den2g = jnp.transpose(den2.reshape(G, HEADS, NP), (0, 2, 1))
    xw, aux, mi = _prep3(msg2, den2g, b2.reshape(1, HEADS * HID), W3, as3,
                         ad3, HEADS)
    asr, ads, di = split_aux(aux, 1)
    den3 = _edgew(s3, d3, asr.reshape(G, NP), ads.reshape(G, NP),
                  di.reshape(G, NP), G)
    msg3 = gat3(xw, mi, asr.reshape(G // 4, 4 * NP),
                ads.reshape(G // 4, 4 * NP), s_flat, d_flat)

    mu, lv, pi = _tail(msg3, den3.reshape(G // 4, 4, NP), b3, pool_W, mask,
                       Wih_f, Whh_f, bih_f, bhh_f, Wih_r, Whh_r, bih_r, bhh_r,
                       Wmu, bmu, Wlv, blv, Wpi, bpi)
    return mu.reshape(B, K, LAT), lv.reshape(B, K, LAT), pi


# speedup vs baseline: 166.8769x; 1.2123x over previous
import functools

import jax
import jax.numpy as jnp
from jax import lax
from jax.experimental import pallas as pl
from jax.experimental.pallas import tpu as pltpu
from jax.experimental.pallas import tpu_sc as plsc

F32 = jnp.float32
I32 = jnp.int32

B, T, N, FD = 4, 8, 1000, 128
HID, HEADS, RNN, LAT, K = 32, 4, 128, 64, 32
G = B * T
NP = 1008
NE = 16000
CHE = 160
EPP = NE // 4
CPP = EPP // CHE
NPR = CPP // 2
ECH = 640
ENCH = NE // ECH
PREC = None


def _head_sel(heads, fout):
    return (lax.broadcasted_iota(I32, (heads, fout), 1) // HID ==
            lax.broadcasted_iota(I32, (heads, fout), 0)).astype(F32)


def _prep_common(x, W_ref, as_ref, ad_ref, heads):
    fout = heads * HID
    xw = jnp.dot(x, W_ref[...], preferred_element_type=F32, precision=PREC)
    as_cat = jnp.concatenate([as_ref[h] for h in range(heads)])
    ad_cat = jnp.concatenate([ad_ref[h] for h in range(heads)])
    rows = lax.broadcasted_iota(I32, (fout, 2 * heads), 0) // HID
    cols = lax.broadcasted_iota(I32, (fout, 2 * heads), 1)
    pick = jnp.where(cols < heads, as_cat[:, None], ad_cat[:, None])
    A = jnp.where(rows == jnp.where(cols < heads, cols, cols - heads),
                  pick, 0.0)
    aa = jnp.dot(xw, A, preferred_element_type=F32, precision=PREC)
    e = aa[:, :heads] + aa[:, heads:]
    wself = jnp.exp(jnp.maximum(e, 0.2 * e))
    wexp = jnp.dot(wself, _head_sel(heads, fout),
                   preferred_element_type=F32, precision=PREC)
    msgi = xw * wexp
    aux = jnp.concatenate([aa, wself], axis=1)
    return xw, aux, msgi


def _normalize(msg, dn, bias_ref, heads_in):
    dn_exp = jnp.dot(dn, _head_sel(heads_in, heads_in * HID),
                     preferred_element_type=F32, precision=PREC)
    return jnp.maximum(msg / dn_exp + bias_ref[0][None, :], 0.0)


def _prep1_body(x_ref, W_ref, as_ref, ad_ref, xw_ref, aux_ref, msgi_ref, *,
                heads):
    xw, aux, msgi = _prep_common(x_ref[0, 0], W_ref, as_ref, ad_ref, heads)
    xw_ref[0] = xw
    aux_ref[0] = aux
    msgi_ref[0] = msgi


def _prepL_body(msg_ref, den_ref, bias_ref, W_ref, as_ref, ad_ref, xw_ref,
                aux_ref, msgi_ref, *, heads, heads_in):
    x = _normalize(msg_ref[0], den_ref[0], bias_ref, heads_in)
    xw, aux, msgi = _prep_common(x, W_ref, as_ref, ad_ref, heads)
    xw_ref[0] = xw
    aux_ref[0] = aux
    msgi_ref[0] = msgi


def _prep3_body(msg_ref, den_ref, bias_ref, W_ref, as_ref, ad_ref, xw_ref,
                aux_ref, msgi_ref, *, heads_in):
    for r in range(4):
        x = _normalize(msg_ref[r], den_ref[r], bias_ref, heads_in)
        xw, aux, msgi = _prep_common(x, W_ref, as_ref, ad_ref, 1)
        xw_ref[0, :, r * HID:(r + 1) * HID] = xw
        msgi_ref[0, :, r * HID:(r + 1) * HID] = msgi
        aux_ref[r] = aux


def _prep1(x_p, W, a_s, a_d, heads):
    fin = W.shape[0]
    return pl.pallas_call(
        functools.partial(_prep1_body, heads=heads),
        grid=(G,),
        in_specs=[
            pl.BlockSpec((1, 1, NP, fin), lambda g: (g, 0, 0, 0)),
            pl.BlockSpec((fin, heads * HID), lambda g: (0, 0)),
            pl.BlockSpec((heads, HID), lambda g: (0, 0)),
            pl.BlockSpec((heads, HID), lambda g: (0, 0)),
        ],
        out_specs=(
            pl.BlockSpec((1, NP, heads * HID), lambda g: (g, 0, 0)),
            pl.BlockSpec((1, NP, 3 * heads), lambda g: (g, 0, 0)),
            pl.BlockSpec((1, NP, heads * HID), lambda g: (g, 0, 0)),
        ),
        out_shape=(
            jax.ShapeDtypeStruct((G, NP, heads * HID), F32),
            jax.ShapeDtypeStruct((G, NP, 3 * heads), F32),
            jax.ShapeDtypeStruct((G, NP, heads * HID), F32),
        ),
    )(x_p, W, a_s, a_d)


def _prepL(msg, den, bias, W, a_s, a_d, heads, heads_in):
    fin = W.shape[0]
    return pl.pallas_call(
        functools.partial(_prepL_body, heads=heads, heads_in=heads_in),
        grid=(G,),
        in_specs=[
            pl.BlockSpec((1, NP, heads_in * HID), lambda g: (g, 0, 0)),
            pl.BlockSpec((1, NP, heads_in), lambda g: (g, 0, 0)),
            pl.BlockSpec((1, heads_in * HID), lambda g: (0, 0)),
            pl.BlockSpec((fin, heads * HID), lambda g: (0, 0)),
            pl.BlockSpec((heads, HID), lambda g: (0, 0)),
            pl.BlockSpec((heads, HID), lambda g: (0, 0)),
        ],
        out_specs=(
            pl.BlockSpec((1, NP, heads * HID), lambda g: (g, 0, 0)),
            pl.BlockSpec((1, NP, 3 * heads), lambda g: (g, 0, 0)),
            pl.BlockSpec((1, NP, heads * HID), lambda g: (g, 0, 0)),
        ),
        out_shape=(
            jax.ShapeDtypeStruct((G, NP, heads * HID), F32),
            jax.ShapeDtypeStruct((G, NP, 3 * heads), F32),
            jax.ShapeDtypeStruct((G, NP, heads * HID), F32),
        ),
    )(msg, den, bias, W, a_s, a_d)


def _prep3(msg, den, bias, W, a_s, a_d, heads_in):
    fin = W.shape[0]
    return pl.pallas_call(
        functools.partial(_prep3_body, heads_in=heads_in),
        grid=(G // 4,),
        in_specs=[
            pl.BlockSpec((4, NP, heads_in * HID), lambda q: (q, 0, 0)),
            pl.BlockSpec((4, NP, heads_in), lambda q: (q, 0, 0)),
            pl.BlockSpec((1, heads_in * HID), lambda q: (0, 0)),
            pl.BlockSpec((fin, HID), lambda q: (0, 0)),
            pl.BlockSpec((1, HID), lambda q: (0, 0)),
            pl.BlockSpec((1, HID), lambda q: (0, 0)),
        ],
        out_specs=(
            pl.BlockSpec((1, NP, 4 * HID), lambda q: (q, 0, 0)),
            pl.BlockSpec((4, NP, 3), lambda q: (q, 0, 0)),
            pl.BlockSpec((1, NP, 4 * HID), lambda q: (q, 0, 0)),
        ),
        out_shape=(
            jax.ShapeDtypeStruct((G // 4, NP, 4 * HID), F32),
            jax.ShapeDtypeStruct((G, NP, 3), F32),
            jax.ShapeDtypeStruct((G // 4, NP, 4 * HID), F32),
        ),
    )(msg, den, bias, W, a_s, a_d)


def _edgew_body(s_ref, d_ref, asrc_ref, adst_ref, deni_ref, den_ref):
    ch = pl.program_id(0)
    sv = s_ref[0, 0]
    dv = d_ref[0, 0]
    nodes = lax.broadcasted_iota(I32, (ECH, NP), 1)
    oh_s = (sv[:, None] == nodes).astype(F32)
    oh_d = (dv[:, None] == nodes).astype(F32)
    asrc_e = lax.dot_general(asrc_ref[...], oh_s, (((1,), (1,)), ((), ())),
                             preferred_element_type=F32, precision=PREC)
    adst_e = lax.dot_general(adst_ref[...], oh_d, (((1,), (1,)), ((), ())),
                             preferred_element_type=F32, precision=PREC)
    e = asrc_e + adst_e
    w = jnp.exp(jnp.maximum(e, 0.2 * e))

    @pl.when(ch == 0)
    def _():
        den_ref[...] = deni_ref[...]

    den_ref[...] += lax.dot_general(w, oh_d, (((1,), (0,)), ((), ())),
                                    preferred_element_type=F32, precision=PREC)


def _edgew(s3, d3, asrc, adst, deni, P):
    return pl.pallas_call(
        _edgew_body,
        grid=(ENCH,),
        in_specs=[
            pl.BlockSpec((1, 1, ECH), lambda ch: (ch, 0, 0)),
            pl.BlockSpec((1, 1, ECH), lambda ch: (ch, 0, 0)),
            pl.BlockSpec((P, NP), lambda ch: (0, 0)),
            pl.BlockSpec((P, NP), lambda ch: (0, 0)),
            pl.BlockSpec((P, NP), lambda ch: (0, 0)),
        ],
        out_specs=pl.BlockSpec((P, NP), lambda ch: (0, 0)),
        out_shape=jax.ShapeDtypeStruct((P, NP), F32),
    )(s3, d3, asrc, adst, deni)


def _scale_rows(tmp_v, w_ch):
    def grp(j, c2):
        base = j * 16
        wvecs = [w_ch[h, pl.ds(base, 16)] for h in range(4)]
        for lane in range(16):
            e = base + lane
            for h in range(4):
                ws = wvecs[h][lane]
                tmp_v[e, pl.ds(h * 32, 16)] = tmp_v[e, pl.ds(h * 32, 16)] * ws
                tmp_v[e, pl.ds(h * 32 + 16, 16)] = (
                    tmp_v[e, pl.ds(h * 32 + 16, 16)] * ws)
        return c2

    lax.fori_loop(0, CHE // 16, grp, 0)


def _wgrp(k, s_v, d_v, w_ch, asrc_v, adst_v):
    def grp(kk, c2):
        sv = s_v[pl.ds(k * CHE + kk * 16, 16)]
        dv = d_v[pl.ds(k * CHE + kk * 16, 16)]
        for h in range(4):
            a1 = plsc.load_gather(asrc_v, [sv + h * NP])
            a2 = plsc.load_gather(adst_v, [dv + h * NP])
            e = a1 + a2
            w_ch[h, pl.ds(kk * 16, 16)] = jnp.exp(jnp.maximum(e, 0.2 * e))
        return c2

    lax.fori_loop(0, CHE // 16, grp, 0)


def _gat_sc(NOBJ):
    NPASS = NOBJ // 8
    mesh = plsc.VectorSubcoreMesh(core_axis_name="c", subcore_axis_name="s",
                                  num_cores=2, num_subcores=16)

    @functools.partial(
        pl.kernel,
        out_type=jax.ShapeDtypeStruct((NOBJ, NP, 4 * HID), F32),
        mesh=mesh,
        compiler_params=pltpu.CompilerParams(needs_layout_passes=False),
        scratch_types=[
            pltpu.VMEM_SHARED((4, NP, 4 * HID), F32),
            pltpu.VMEM((EPP,), I32),
            pltpu.VMEM((EPP,), I32),
            pltpu.VMEM((4, CHE), F32),
            pltpu.VMEM((4, CHE), F32),
            pltpu.VMEM((4 * NP,), F32),
            pltpu.VMEM((4 * NP,), F32),
            pltpu.VMEM((CHE, 4 * HID), F32),
            pltpu.VMEM((CHE, 4 * HID), F32),
            pltpu.SemaphoreType.DMA,
            pltpu.SemaphoreType.DMA,
        ],
    )
    def k(xw_hbm, msgi_hbm, asrc_hbm, adst_hbm, sf_hbm, df_hbm, msg_out,
          msg_spm, s_v, d_v, w_a, w_b, asrc_v, adst_v, tmp_a, tmp_b,
          sem_a, sem_b):
        cid = lax.axis_index("c")
        sid = lax.axis_index("s")
        reg = sid // 4
        part = sid % 4
        pltpu.sync_copy(sf_hbm.at[pl.ds(part * EPP, EPP)], s_v)
        pltpu.sync_copy(df_hbm.at[pl.ds(part * EPP, EPP)], d_v)

        def gstart(obj, k_local, tmp, sem):
            idx = s_v.at[pl.ds(k_local * CHE, CHE)]
            pltpu.make_async_copy(xw_hbm.at[obj].at[idx], tmp, sem).start()

        def gwait(obj, k_local, tmp, sem):
            idx = s_v.at[pl.ds(k_local * CHE, CHE)]
            pltpu.make_async_copy(xw_hbm.at[obj].at[idx], tmp, sem).wait()

        def process(obj, k_local, tmp, w_ch, sem):
            _wgrp(k_local, s_v, d_v, w_ch, asrc_v, adst_v)
            gwait(obj, k_local, tmp, sem)
            _scale_rows(tmp, w_ch)
            idx = d_v.at[pl.ds(k_local * CHE, CHE)]
            pltpu.sync_copy(tmp, msg_spm.at[reg].at[idx], add=True)

        for pp in range(NPASS):
            obj = pp * 8 + cid * 4 + reg
            pltpu.sync_copy(asrc_hbm.at[obj], asrc_v)
            pltpu.sync_copy(adst_hbm.at[obj], adst_v)

            @pl.when(part == 0)
            def _():
                pltpu.sync_copy(msgi_hbm.at[obj], msg_spm.at[reg])

            plsc.subcore_barrier()
            gstart(obj, 0, tmp_a, sem_a)

            def pair(jj, carry):
                c0 = 2 * jj
                gstart(obj, c0 + 1, tmp_b, sem_b)
                process(obj, c0, tmp_a, w_a, sem_a)

                @pl.when(c0 + 2 < CPP)
                def _():
                    gstart(obj, c0 + 2, tmp_a, sem_a)

                process(obj, c0 + 1, tmp_b, w_b, sem_b)
                return carry

            lax.fori_loop(0, NPR, pair, 0)
            if CPP % 2:
                process(obj, CPP - 1, tmp_a, w_a, sem_a)
            plsc.subcore_barrier()

            @pl.when(part < 3)
            def _():
                pltpu.sync_copy(msg_spm.at[reg].at[pl.ds(part * 256, 256)],
                                msg_out.at[obj].at[pl.ds(part * 256, 256)])

            @pl.when(part == 3)
            def _():
                pltpu.sync_copy(msg_spm.at[reg].at[pl.ds(768, NP - 768)],
                                msg_out.at[obj].at[pl.ds(768, NP - 768)])

            plsc.subcore_barrier()

    return k


def _tail_body(msg3_ref, den3_ref, b3_ref, pw_ref, mask_ref,
               Wih_f_ref, Whh_f_ref, bih_f_ref, bhh_f_ref,
               Wih_r_ref, Whh_r_ref, bih_r_ref, bhh_r_ref,
               Wmu_ref, bmu_ref, Wlv_ref, blv_ref, Wpi_ref, bpi_ref,
               mu_ref, lv_ref, pi_ref):
    pw = pw_ref[...][:, 0]
    valid = lax.broadcasted_iota(I32, (1, NP), 1) < N
    pooled_parts = []
    for r in range(4):
        m = msg3_ref[...][:, :, r * HID:(r + 1) * HID]
        dn = den3_ref[...][:, r, :]
        h3 = jnp.maximum(m / dn[..., None] + b3_ref[...][None, None, :], 0.0)
        logits = jnp.sum(h3 * pw[None, None, :], axis=-1)
        ex = jnp.where(valid, jnp.exp(logits), 0.0)
        denp = jnp.sum(ex, axis=-1)
        pooled_parts.append(
            jnp.sum(ex[..., None] * h3, axis=1) / (denp[:, None] + 1e-16))
    pooled = jnp.stack(pooled_parts, axis=1).reshape(G, HID)
    mask = mask_ref[...]
    ge = pooled.reshape(B, T, HID) * mask
    lengths = jnp.clip(jnp.sum(mask[:, :, 0], axis=1), 1, None).astype(I32)

    def lstm(Wih, Whh, bih, bhh, reverse):
        h = jnp.zeros((B, RNN), F32)
        c = jnp.zeros((B, RNN), F32)
        for kk in range(T):
            t = T - 1 - kk if reverse else kk
            xt = ge[:, t, :]
            g = (lax.dot_general(xt, Wih, (((1,), (1,)), ((), ())),
                                 precision=PREC) + bih[None, :] +
                 lax.dot_general(h, Whh, (((1,), (1,)), ((), ())),
                                 precision=PREC) + bhh[None, :])
            i, f, gg, o = jnp.split(g, 4, axis=-1)
            i = jax.nn.sigmoid(i)
            f = jax.nn.sigmoid(f)
            gg = jnp.tanh(gg)
            o = jax.nn.sigmoid(o)
            cn = f * c + i * gg
            hn = o * jnp.tanh(cn)
            ok = (t < lengths)[:, None]
            h = jnp.where(ok, hn, h)
            c = jnp.where(ok, cn, c)
        return h

    hf = lstm(Wih_f_ref[...], Whh_f_ref[...], bih_f_ref[...], bhh_f_ref[...],
              False)
    hr = lstm(Wih_r_ref[...], Whh_r_ref[...], bih_r_ref[...], bhh_r_ref[...],
              True)
    feat = jnp.concatenate([hf, hr], axis=1)
    mu_ref[...] = lax.dot_general(feat, Wmu_ref[...], (((1,), (1,)), ((), ())),
                                  precision=PREC) + bmu_ref[...][None, :]
    lv_ref[...] = lax.dot_general(feat, Wlv_ref[...], (((1,), (1,)), ((), ())),
                                  precision=PREC) + blv_ref[...][None, :]
    pi_ref[...] = lax.dot_general(feat, Wpi_ref[...], (((1,), (1,)), ((), ())),
                                  precision=PREC) + bpi_ref[...][None, :]


def _tail(msg3, den3, b3, pool_W, mask, Wih_f, Whh_f, bih_f, bhh_f,
          Wih_r, Whh_r, bih_r, bhh_r, Wmu, bmu, Wlv, blv, Wpi, bpi):
    return pl.pallas_call(
        _tail_body,
        out_shape=(
            jax.ShapeDtypeStruct((B, K * LAT), F32),
            jax.ShapeDtypeStruct((B, K * LAT), F32),
            jax.ShapeDtypeStruct((B, K), F32),
        ),
    )(msg3, den3, b3, pool_W, mask, Wih_f, Whh_f, bih_f, bhh_f,
      Wih_r, Whh_r, bih_r, bhh_r, Wmu, bmu, Wlv, blv, Wpi, bpi)


def kernel(x, edge_index, mask, W1, as1, ad1, b1, W2, as2, ad2, b2, W3, as3,
           ad3, b3, pool_W, pool_b, Wih_f, Whh_f, bih_f, bhh_f, Wih_r, Whh_r,
           bih_r, bhh_r, Wmu, bmu, Wlv, blv, Wpi, bpi):
    del pool_b
    x_p = jnp.pad(x.reshape(G, N, FD), ((0, 0), (0, NP - N), (0, 0)))
    x_p = x_p.reshape(G, 1, NP, FD)
    s_flat = edge_index[0]
    d_flat = edge_index[1]
    s3 = s_flat.reshape(ENCH, 1, ECH)
    d3 = d_flat.reshape(ENCH, 1, ECH)

    gat12 = _gat_sc(G)
    gat3 = _gat_sc(G // 4)
    P = G * HEADS

    def split_aux(aux, heads):
        auxt = jnp.transpose(aux, (0, 2, 1))
        return (auxt[:, :heads], auxt[:, heads:2 * heads],
                auxt[:, 2 * heads:])

    xw, aux, mi = _prep1(x_p, W1, as1, ad1, HEADS)
    asr, ads, di = split_aux(aux, HEADS)
    den1 = _edgew(s3, d3, asr.reshape(P, NP), ads.reshape(P, NP),
                  di.reshape(P, NP), P)
    msg1 = gat12(xw, mi, asr.reshape(G, 4 * NP), ads.reshape(G, 4 * NP),
                 s_flat, d_flat)
    den1g = jnp.transpose(den1.reshape(G, HEADS, NP), (0, 2, 1))
    xw, aux, mi = _prepL(msg1, den1g, b1.reshape(1, HEADS * HID), W2, as2,
                         ad2, HEADS, HEADS)
    asr, ads, di = split_aux(aux, HEADS)
    den2 = _edgew(s3, d3, asr.reshape(P, NP), ads.reshape(P, NP),
                  di.reshape(P, NP), P)
    msg2 = gat12(xw, mi, asr.reshape(G, 4 * NP), ads.reshape(G, 4 * NP),
                 s_flat, d_flat)
    den2g = jnp.transpose(den2.reshape(G, HEADS, NP), (0, 2, 1))
    xw, aux, mi = _prep3(msg2, den2g, b2.reshape(1, HEADS * HID), W3, as3,
                         ad3, HEADS)
    asr, ads, di = split_aux(aux, 1)
    den3 = _edgew(s3, d3, asr.reshape(G, NP), ads.reshape(G, NP),
                  di.reshape(G, NP), G)
    msg3 = gat3(xw, mi, asr.reshape(G // 4, 4 * NP),
                ads.reshape(G // 4, 4 * NP), s_flat, d_flat)

    mu, lv, pi = _tail(msg3, den3.reshape(G // 4, 4, NP), b3, pool_W, mask,
                       Wih_f, Whh_f, bih_f, bhh_f, Wih_r, Whh_r, bih_r, bhh_r,
                       Wmu, bmu, Wlv, blv, Wpi, bpi)
    return mu.reshape(B, K, LAT), lv.reshape(B, K, LAT), pi
